# baseline (speedup 1.0000x reference)
import functools

import jax
import jax.numpy as jnp
from jax import lax
from jax.experimental import pallas as pl
from jax.experimental.pallas import tpu as pltpu
from jax.experimental.pallas import tpu_sc as plsc

N = 10000
E = 320000
D = 128
H = 16
G = 128
NC = 2
NS = 16
N_PAD = 10240
ROWS = E // 128
CH = 10
CHUNKS = ROWS // CH
C0_CHUNKS = 150
C1_CHUNKS = CHUNKS - C0_CHUNKS
_Q0, _R0 = divmod(C0_CHUNKS, NS)
_Q1, _R1 = divmod(C1_CHUNKS, NS)
R_MAX = (max(_Q0 + (1 if _R0 else 0), _Q1 + (1 if _R1 else 0))) * CH
CHE = CH * 128
RPT = N_PAD // NS
NP8 = N // 8
PADP = N_PAD * H // 128


def _worker_span(c, s):
    on0 = (c == 0)
    nch = jnp.where(on0, _Q0 + (s < _R0), _Q1 + (s < _R1)).astype(jnp.int32)
    cb = jnp.where(on0, s * _Q0 + jnp.minimum(s, _R0),
                   C0_CHUNKS + s * _Q1 + jnp.minimum(s, _R1)).astype(jnp.int32)
    return cb * CH, nch


def _deg_body(adj_hbm, out_hbm, idx_v, ones_v, zero_v, deg_sh, sem_s):
    c = lax.axis_index("c")
    s = lax.axis_index("s")
    for i in range(8):
        ones_v[pl.ds(i * 16, 16)] = jnp.ones((16,), jnp.float32)

    def zinit(i, carry):
        zero_v[pl.ds(i * 16, 16)] = jnp.zeros((16,), jnp.float32)
        return carry

    lax.fori_loop(0, RPT // 16, zinit, 0)
    pltpu.sync_copy(zero_v, deg_sh.at[pl.ds(s * RPT, RPT)])
    plsc.subcore_barrier()

    base, nch = _worker_span(c, s)
    pstart = jnp.minimum(base, ROWS - R_MAX)
    d = base - pstart
    pltpu.sync_copy(adj_hbm.at[1, pl.ds(pstart, R_MAX)], idx_v)
    drain = pltpu.make_async_copy(
        adj_hbm.at[1, pl.ds(0, CH)], idx_v.at[pl.ds(0, CH)], sem_s)

    def step(j, carry):
        @pl.when(j >= 1)
        def _():
            drain.wait()

        for k in range(CH):
            pltpu.async_copy(ones_v, deg_sh.at[idx_v.at[d + j * CH + k]],
                             sem_s, add=True)
        return carry

    lax.fori_loop(0, nch, step, 0)
    drain.wait()
    plsc.subcore_barrier()
    pltpu.sync_copy(deg_sh.at[pl.ds(s * RPT, RPT)],
                    out_hbm.at[c, pl.ds(s * RPT, RPT)])


def _scat_body(tab_hbm, adj_hbm, out_hbm,
               idx_v, rows_v, zrow_v, acc_sh, sem_g, sem_s):
    c = lax.axis_index("c")
    s = lax.axis_index("s")

    def zinit(i, carry):
        zrow_v[i, :] = jnp.zeros((16,), jnp.float32)
        return carry

    lax.fori_loop(0, RPT, zinit, 0)
    pltpu.sync_copy(zrow_v, acc_sh.at[pl.ds(s * RPT, RPT)])
    plsc.subcore_barrier()

    base, nch = _worker_span(c, s)
    pstart = jnp.minimum(base, ROWS - R_MAX)
    d = base - pstart
    pltpu.sync_copy(adj_hbm.at[0, pl.ds(pstart, R_MAX)], idx_v.at[0])
    pltpu.sync_copy(adj_hbm.at[1, pl.ds(pstart, R_MAX)], idx_v.at[1])

    dummy = tab_hbm.at[pl.ds(0, CHE)]

    def fire_gathers(j, b):
        for k in range(CH):
            pltpu.async_copy(tab_hbm.at[idx_v.at[0, d + j * CH + k]],
                             rows_v.at[b, pl.ds(k * 128, 128)], sem_g)

    def fire_scatters(j, b):
        for k in range(CH):
            pltpu.async_copy(rows_v.at[b, pl.ds(k * 128, 128)],
                             acc_sh.at[idx_v.at[1, d + j * CH + k]],
                             sem_s, add=True)

    fire_gathers(0, 0)

    def step(j, carry):
        b = lax.rem(j, 2)

        @pl.when(j >= 1)
        def _():
            pltpu.make_async_copy(dummy, rows_v.at[1 - b], sem_s).wait()

        @pl.when(j < nch - 1)
        def _():
            fire_gathers(j + 1, 1 - b)

        pltpu.make_async_copy(dummy, rows_v.at[b], sem_g).wait()
        fire_scatters(j, b)
        return carry

    lax.fori_loop(0, nch, step, 0)
    pltpu.make_async_copy(dummy, rows_v.at[lax.rem(nch - 1, 2)], sem_s).wait()
    plsc.subcore_barrier()
    pltpu.sync_copy(acc_sh.at[pl.ds(s * RPT, RPT)],
                    out_hbm.at[c, pl.ds(s * RPT, RPT)])


@functools.cache
def _sc_calls():
    mesh = plsc.VectorSubcoreMesh(core_axis_name="c", subcore_axis_name="s")
    params = pltpu.CompilerParams(use_tc_tiling_on_sc=False)
    deg_call = pl.kernel(
        _deg_body,
        out_type=jax.ShapeDtypeStruct((NC, N_PAD), jnp.float32),
        mesh=mesh,
        scratch_types=[
            pltpu.VMEM((R_MAX, 128), jnp.int32),
            pltpu.VMEM((128,), jnp.float32),
            pltpu.VMEM((RPT,), jnp.float32),
            pltpu.VMEM_SHARED((N_PAD,), jnp.float32),
            pltpu.SemaphoreType.DMA,
        ],
        compiler_params=params,
    )
    scat_call = pl.kernel(
        _scat_body,
        out_type=jax.ShapeDtypeStruct((NC, N_PAD, H), jnp.float32),
        mesh=mesh,
        scratch_types=[
            pltpu.VMEM((2, R_MAX, 128), jnp.int32),
            pltpu.VMEM((2, CHE, H), jnp.float32),
            pltpu.VMEM((RPT, H), jnp.float32),
            pltpu.VMEM_SHARED((N_PAD, H), jnp.float32),
            pltpu.SemaphoreType.DMA,
            pltpu.SemaphoreType.DMA,
        ],
        compiler_params=params,
    )
    return deg_call, scat_call


def _tc1_body(x_ref, w1_ref, degp_ref, hp_ref, dinvb_ref):
    deg = degp_ref[0] + degp_ref[1] + 1.0
    dinv = lax.rsqrt(deg)[:N].reshape(N, 1)
    h = jnp.dot(x_ref[...], w1_ref[...], preferred_element_type=jnp.float32)
    hp_ref[...] = h * dinv
    dinvb_ref[...] = jnp.broadcast_to(dinv, (N, H))


_tc1_call = pl.pallas_call(
    _tc1_body,
    out_shape=[
        jax.ShapeDtypeStruct((N, H), jnp.float32),
        jax.ShapeDtypeStruct((N, H), jnp.float32),
    ],
)


def _tc2_body(acc_ref, hp_ref, dinvp_ref, b1t_ref, w2b_ref, hp2_ref):
    a = acc_ref[...]
    accp = a[:NP8] + a[PADP:PADP + NP8]
    dinvp = dinvp_ref[...]
    x1 = jnp.maximum(dinvp * (accp + hp_ref[...]) + b1t_ref[...], 0.0)
    h2 = jnp.dot(x1, w2b_ref[...], preferred_element_type=jnp.float32)
    hp2_ref[...] = h2 * dinvp


_tc2_call = pl.pallas_call(
    _tc2_body,
    out_shape=jax.ShapeDtypeStruct((NP8, 128), jnp.float32),
)


def _tc3_body(acc_ref, hp2_ref, dinvp_ref, b2t_ref, batch_ref,
              wfc_ref, bfc_ref, wout_ref, bout_ref, out_ref):
    a = acc_ref[...]
    accp = a[:NP8] + a[PADP:PADP + NP8]
    x2p = jnp.maximum(dinvp_ref[...] * (accp + hp2_ref[...]) + b2t_ref[...],
                      0.0)
    bt = batch_ref[...]
    gi = lax.broadcasted_iota(jnp.int32, (NP8, G), 1)
    ones_col = jnp.ones((NP8, 1), jnp.float32)
    dn = (((0,), (0,)), ((), ()))
    seg = jnp.zeros((G, H), jnp.float32)
    cnt = jnp.zeros((G, 1), jnp.float32)
    for j in range(8):
        mj = jnp.where(bt[:, j:j + 1] == gi, 1.0, 0.0)
        xs = x2p[:, H * j:H * (j + 1)]
        seg = seg + lax.dot_general(mj, xs, dn,
                                    preferred_element_type=jnp.float32)
        cnt = cnt + lax.dot_general(mj, ones_col, dn,
                                    preferred_element_type=jnp.float32)
    pooled = seg / jnp.maximum(cnt, 1.0)
    hfc = jnp.maximum(
        jnp.dot(pooled, wfc_ref[...], preferred_element_type=jnp.float32)
        + bfc_ref[...], 0.0)
    out_ref[...] = (jnp.dot(hfc, wout_ref[...],
                            preferred_element_type=jnp.float32) + bout_ref[...])


_tc3_call = pl.pallas_call(
    _tc3_body,
    out_shape=jax.ShapeDtypeStruct((G, 2), jnp.float32),
)


def kernel(features, adj, batch, W1, b1, W2, b2, Wfc, bfc, Wout, bout):
    adjr = adj.reshape(2, ROWS, 128)
    w2big = (jnp.eye(8, dtype=jnp.float32)[:, None, :, None]
             * W2[None, :, None, :]).reshape(8 * H, 8 * H)
    b1t = jnp.tile(b1, 8).reshape(1, 128)
    b2t = jnp.tile(b2, 8).reshape(1, 128)

    deg_call, scat_call = _sc_calls()
    degp = deg_call(adjr)
    hp1, dinvb = _tc1_call(features, W1, degp)
    dinvp = dinvb.reshape(NP8, 128)
    acc1 = scat_call(hp1, adjr)
    hp2p = _tc2_call(acc1.reshape(2 * PADP, 128), hp1.reshape(NP8, 128),
                     dinvp, b1t, w2big)
    acc2 = scat_call(hp2p.reshape(N, H), adjr)
    out = _tc3_call(acc2.reshape(2 * PADP, 128), hp2p, dinvp, b2t,
                    batch.reshape(NP8, 8),
                    Wfc, bfc.reshape(1, 16), Wout, bout.reshape(1, 2))
    return out


# speedup vs baseline: 1.0393x; 1.0393x over previous
import functools

import jax
import jax.numpy as jnp
from jax import lax
from jax.experimental import pallas as pl
from jax.experimental.pallas import tpu as pltpu
from jax.experimental.pallas import tpu_sc as plsc

N = 10000
E = 320000
D = 128
H = 16
G = 128
NC = 2
NS = 16
N_PAD = 10240
ROWS = E // 128
CH = 10
CHUNKS = ROWS // CH
C0_CHUNKS = 140
C1_CHUNKS = CHUNKS - C0_CHUNKS
_Q0, _R0 = divmod(C0_CHUNKS, NS)
_Q1, _R1 = divmod(C1_CHUNKS, NS)
R_MAX = (max(_Q0 + (1 if _R0 else 0), _Q1 + (1 if _R1 else 0))) * CH
CHE = CH * 128
RPT = N_PAD // NS
NP8 = N // 8
PADP = N_PAD * H // 128


def _worker_span(c, s):
    on0 = (c == 0)
    nch = jnp.where(on0, _Q0 + (s < _R0), _Q1 + (s < _R1)).astype(jnp.int32)
    cb = jnp.where(on0, s * _Q0 + jnp.minimum(s, _R0),
                   C0_CHUNKS + s * _Q1 + jnp.minimum(s, _R1)).astype(jnp.int32)
    return cb * CH, nch


def _deg_body(adj_hbm, out_hbm, idx_v, ones_v, zero_v, deg_sh, sem_s):
    c = lax.axis_index("c")
    s = lax.axis_index("s")
    for i in range(8):
        ones_v[pl.ds(i * 16, 16)] = jnp.ones((16,), jnp.float32)

    def zinit(i, carry):
        zero_v[pl.ds(i * 16, 16)] = jnp.zeros((16,), jnp.float32)
        return carry

    lax.fori_loop(0, RPT // 16, zinit, 0)
    pltpu.sync_copy(zero_v, deg_sh.at[pl.ds(s * RPT, RPT)])
    plsc.subcore_barrier()

    base, nch = _worker_span(c, s)
    pstart = jnp.minimum(base, ROWS - R_MAX)
    d = base - pstart
    pltpu.sync_copy(adj_hbm.at[1, pl.ds(pstart, R_MAX)], idx_v)
    drain = pltpu.make_async_copy(
        adj_hbm.at[1, pl.ds(0, CH)], idx_v.at[pl.ds(0, CH)], sem_s)

    def step(j, carry):
        @pl.when(j >= 1)
        def _():
            drain.wait()

        for k in range(CH):
            pltpu.async_copy(ones_v, deg_sh.at[idx_v.at[d + j * CH + k]],
                             sem_s, add=True)
        return carry

    lax.fori_loop(0, nch, step, 0)
    drain.wait()
    plsc.subcore_barrier()
    pltpu.sync_copy(deg_sh.at[pl.ds(s * RPT, RPT)],
                    out_hbm.at[c, pl.ds(s * RPT, RPT)])


def _scat_body(tab_hbm, adj_hbm, out_hbm,
               idx_v, rows_v, zrow_v, acc_sh, sem_g, sem_s):
    c = lax.axis_index("c")
    s = lax.axis_index("s")

    def zinit(i, carry):
        zrow_v[i, :] = jnp.zeros((16,), jnp.float32)
        return carry

    lax.fori_loop(0, RPT, zinit, 0)
    pltpu.sync_copy(zrow_v, acc_sh.at[pl.ds(s * RPT, RPT)])
    plsc.subcore_barrier()

    base, nch = _worker_span(c, s)
    pstart = jnp.minimum(base, ROWS - R_MAX)
    d = base - pstart
    pltpu.sync_copy(adj_hbm.at[0, pl.ds(pstart, R_MAX)], idx_v.at[0])
    pltpu.sync_copy(adj_hbm.at[1, pl.ds(pstart, R_MAX)], idx_v.at[1])

    dummy = tab_hbm.at[pl.ds(0, CHE)]

    def fire_gathers(j, b):
        for k in range(CH):
            pltpu.async_copy(tab_hbm.at[idx_v.at[0, d + j * CH + k]],
                             rows_v.at[b, pl.ds(k * 128, 128)], sem_g)

    def fire_scatters(j, b):
        for k in range(CH):
            pltpu.async_copy(rows_v.at[b, pl.ds(k * 128, 128)],
                             acc_sh.at[idx_v.at[1, d + j * CH + k]],
                             sem_s, add=True)

    fire_gathers(0, 0)

    def step(j, carry):
        b = lax.rem(j, 2)

        @pl.when(j >= 1)
        def _():
            pltpu.make_async_copy(dummy, rows_v.at[1 - b], sem_s).wait()

        @pl.when(j < nch - 1)
        def _():
            fire_gathers(j + 1, 1 - b)

        pltpu.make_async_copy(dummy, rows_v.at[b], sem_g).wait()
        fire_scatters(j, b)
        return carry

    lax.fori_loop(0, nch, step, 0)
    pltpu.make_async_copy(dummy, rows_v.at[lax.rem(nch - 1, 2)], sem_s).wait()
    plsc.subcore_barrier()
    pltpu.sync_copy(acc_sh.at[pl.ds(s * RPT, RPT)],
                    out_hbm.at[c, pl.ds(s * RPT, RPT)])


@functools.cache
def _sc_calls():
    mesh = plsc.VectorSubcoreMesh(core_axis_name="c", subcore_axis_name="s")
    params = pltpu.CompilerParams(use_tc_tiling_on_sc=False)
    deg_call = pl.kernel(
        _deg_body,
        out_type=jax.ShapeDtypeStruct((NC, N_PAD), jnp.float32),
        mesh=mesh,
        scratch_types=[
            pltpu.VMEM((R_MAX, 128), jnp.int32),
            pltpu.VMEM((128,), jnp.float32),
            pltpu.VMEM((RPT,), jnp.float32),
            pltpu.VMEM_SHARED((N_PAD,), jnp.float32),
            pltpu.SemaphoreType.DMA,
        ],
        compiler_params=params,
    )
    scat_call = pl.kernel(
        _scat_body,
        out_type=jax.ShapeDtypeStruct((NC, N_PAD, H), jnp.float32),
        mesh=mesh,
        scratch_types=[
            pltpu.VMEM((2, R_MAX, 128), jnp.int32),
            pltpu.VMEM((2, CHE, H), jnp.float32),
            pltpu.VMEM((RPT, H), jnp.float32),
            pltpu.VMEM_SHARED((N_PAD, H), jnp.float32),
            pltpu.SemaphoreType.DMA,
            pltpu.SemaphoreType.DMA,
        ],
        compiler_params=params,
    )
    return deg_call, scat_call


def _tc1_body(x_ref, w1_ref, degp_ref, hp_ref, dinvb_ref):
    deg = degp_ref[0] + degp_ref[1] + 1.0
    dinv = lax.rsqrt(deg)[:N].reshape(N, 1)
    h = jnp.dot(x_ref[...], w1_ref[...], preferred_element_type=jnp.float32)
    hp_ref[...] = h * dinv
    dinvb_ref[...] = jnp.broadcast_to(dinv, (N, H))


_tc1_call = pl.pallas_call(
    _tc1_body,
    out_shape=[
        jax.ShapeDtypeStruct((N, H), jnp.float32),
        jax.ShapeDtypeStruct((N, H), jnp.float32),
    ],
)


def _tc2_body(acc_ref, hp_ref, dinvp_ref, b1t_ref, w2b_ref, hp2_ref):
    a = acc_ref[...]
    accp = a[:NP8] + a[PADP:PADP + NP8]
    dinvp = dinvp_ref[...]
    x1 = jnp.maximum(dinvp * (accp + hp_ref[...]) + b1t_ref[...], 0.0)
    h2 = jnp.dot(x1, w2b_ref[...], preferred_element_type=jnp.float32)
    hp2_ref[...] = h2 * dinvp


_tc2_call = pl.pallas_call(
    _tc2_body,
    out_shape=jax.ShapeDtypeStruct((NP8, 128), jnp.float32),
)


def _tc3_body(acc_ref, hp2_ref, dinvp_ref, b2t_ref, batch_ref,
              wfc_ref, bfc_ref, wout_ref, bout_ref, out_ref):
    a = acc_ref[...]
    accp = a[:NP8] + a[PADP:PADP + NP8]
    x2p = jnp.maximum(dinvp_ref[...] * (accp + hp2_ref[...]) + b2t_ref[...],
                      0.0)
    bt = batch_ref[...]
    gi = lax.broadcasted_iota(jnp.int32, (NP8, G), 1)
    ones_col = jnp.ones((NP8, 1), jnp.float32)
    dn = (((0,), (0,)), ((), ()))
    seg = jnp.zeros((G, H), jnp.float32)
    cnt = jnp.zeros((G, 1), jnp.float32)
    for j in range(8):
        mj = jnp.where(bt[:, j:j + 1] == gi, 1.0, 0.0)
        xs = x2p[:, H * j:H * (j + 1)]
        seg = seg + lax.dot_general(mj, xs, dn,
                                    preferred_element_type=jnp.float32)
        cnt = cnt + lax.dot_general(mj, ones_col, dn,
                                    preferred_element_type=jnp.float32)
    pooled = seg / jnp.maximum(cnt, 1.0)
    hfc = jnp.maximum(
        jnp.dot(pooled, wfc_ref[...], preferred_element_type=jnp.float32)
        + bfc_ref[...], 0.0)
    out_ref[...] = (jnp.dot(hfc, wout_ref[...],
                            preferred_element_type=jnp.float32) + bout_ref[...])


_tc3_call = pl.pallas_call(
    _tc3_body,
    out_shape=jax.ShapeDtypeStruct((G, 2), jnp.float32),
)


def kernel(features, adj, batch, W1, b1, W2, b2, Wfc, bfc, Wout, bout):
    adjr = adj.reshape(2, ROWS, 128)
    w2big = (jnp.eye(8, dtype=jnp.float32)[:, None, :, None]
             * W2[None, :, None, :]).reshape(8 * H, 8 * H)
    b1t = jnp.tile(b1, 8).reshape(1, 128)
    b2t = jnp.tile(b2, 8).reshape(1, 128)

    deg_call, scat_call = _sc_calls()
    degp = deg_call(adjr)
    hp1, dinvb = _tc1_call(features, W1, degp)
    dinvp = dinvb.reshape(NP8, 128)
    acc1 = scat_call(hp1, adjr)
    hp2p = _tc2_call(acc1.reshape(2 * PADP, 128), hp1.reshape(NP8, 128),
                     dinvp, b1t, w2big)
    acc2 = scat_call(hp2p.reshape(N, H), adjr)
    out = _tc3_call(acc2.reshape(2 * PADP, 128), hp2p, dinvp, b2t,
                    batch.reshape(NP8, 8),
                    Wfc, bfc.reshape(1, 16), Wout, bout.reshape(1, 2))
    return out


# speedup vs baseline: 1.0410x; 1.0016x over previous
import functools

import jax
import jax.numpy as jnp
from jax import lax
from jax.experimental import pallas as pl
from jax.experimental.pallas import tpu as pltpu
from jax.experimental.pallas import tpu_sc as plsc

N = 10000
E = 320000
D = 128
H = 16
G = 128
NC = 2
NS = 16
N_PAD = 10240
ROWS = E // 128
CH = 10
CHUNKS = ROWS // CH
C0_CHUNKS = 130
C1_CHUNKS = CHUNKS - C0_CHUNKS
_Q0, _R0 = divmod(C0_CHUNKS, NS)
_Q1, _R1 = divmod(C1_CHUNKS, NS)
R_MAX = (max(_Q0 + (1 if _R0 else 0), _Q1 + (1 if _R1 else 0))) * CH
CHE = CH * 128
RPT = N_PAD // NS
NP8 = N // 8
PADP = N_PAD * H // 128


def _worker_span(c, s):
    on0 = (c == 0)
    nch = jnp.where(on0, _Q0 + (s < _R0), _Q1 + (s < _R1)).astype(jnp.int32)
    cb = jnp.where(on0, s * _Q0 + jnp.minimum(s, _R0),
                   C0_CHUNKS + s * _Q1 + jnp.minimum(s, _R1)).astype(jnp.int32)
    return cb * CH, nch


def _deg_body(adj_hbm, out_hbm, idx_v, ones_v, zero_v, deg_sh, sem_s):
    c = lax.axis_index("c")
    s = lax.axis_index("s")
    for i in range(8):
        ones_v[pl.ds(i * 16, 16)] = jnp.ones((16,), jnp.float32)

    def zinit(i, carry):
        zero_v[pl.ds(i * 16, 16)] = jnp.zeros((16,), jnp.float32)
        return carry

    lax.fori_loop(0, RPT // 16, zinit, 0)
    pltpu.sync_copy(zero_v, deg_sh.at[pl.ds(s * RPT, RPT)])
    plsc.subcore_barrier()

    base, nch = _worker_span(c, s)
    pstart = jnp.minimum(base, ROWS - R_MAX)
    d = base - pstart
    pltpu.sync_copy(adj_hbm.at[1, pl.ds(pstart, R_MAX)], idx_v)
    drain = pltpu.make_async_copy(
        adj_hbm.at[1, pl.ds(0, CH)], idx_v.at[pl.ds(0, CH)], sem_s)

    def step(j, carry):
        @pl.when(j >= 1)
        def _():
            drain.wait()

        for k in range(CH):
            pltpu.async_copy(ones_v, deg_sh.at[idx_v.at[d + j * CH + k]],
                             sem_s, add=True)
        return carry

    lax.fori_loop(0, nch, step, 0)
    drain.wait()
    plsc.subcore_barrier()
    pltpu.sync_copy(deg_sh.at[pl.ds(s * RPT, RPT)],
                    out_hbm.at[c, pl.ds(s * RPT, RPT)])


def _scat_body(tab_hbm, adj_hbm, out_hbm,
               idx_v, rows_v, zrow_v, acc_sh, sem_g, sem_s):
    c = lax.axis_index("c")
    s = lax.axis_index("s")

    def zinit(i, carry):
        zrow_v[i, :] = jnp.zeros((16,), jnp.float32)
        return carry

    lax.fori_loop(0, RPT, zinit, 0)
    pltpu.sync_copy(zrow_v, acc_sh.at[pl.ds(s * RPT, RPT)])
    plsc.subcore_barrier()

    base, nch = _worker_span(c, s)
    pstart = jnp.minimum(base, ROWS - R_MAX)
    d = base - pstart
    pltpu.sync_copy(adj_hbm.at[0, pl.ds(pstart, R_MAX)], idx_v.at[0])
    pltpu.sync_copy(adj_hbm.at[1, pl.ds(pstart, R_MAX)], idx_v.at[1])

    dummy = tab_hbm.at[pl.ds(0, CHE)]

    def fire_gathers(j, b):
        for k in range(CH):
            pltpu.async_copy(tab_hbm.at[idx_v.at[0, d + j * CH + k]],
                             rows_v.at[b, pl.ds(k * 128, 128)], sem_g)

    def fire_scatters(j, b):
        for k in range(CH):
            pltpu.async_copy(rows_v.at[b, pl.ds(k * 128, 128)],
                             acc_sh.at[idx_v.at[1, d + j * CH + k]],
                             sem_s, add=True)

    fire_gathers(0, 0)

    def step(j, carry):
        b = lax.rem(j, 2)

        @pl.when(j >= 1)
        def _():
            pltpu.make_async_copy(dummy, rows_v.at[1 - b], sem_s).wait()

        @pl.when(j < nch - 1)
        def _():
            fire_gathers(j + 1, 1 - b)

        pltpu.make_async_copy(dummy, rows_v.at[b], sem_g).wait()
        fire_scatters(j, b)
        return carry

    lax.fori_loop(0, nch, step, 0)
    pltpu.make_async_copy(dummy, rows_v.at[lax.rem(nch - 1, 2)], sem_s).wait()
    plsc.subcore_barrier()
    pltpu.sync_copy(acc_sh.at[pl.ds(s * RPT, RPT)],
                    out_hbm.at[c, pl.ds(s * RPT, RPT)])


@functools.cache
def _sc_calls():
    mesh = plsc.VectorSubcoreMesh(core_axis_name="c", subcore_axis_name="s")
    params = pltpu.CompilerParams(use_tc_tiling_on_sc=False)
    deg_call = pl.kernel(
        _deg_body,
        out_type=jax.ShapeDtypeStruct((NC, N_PAD), jnp.float32),
        mesh=mesh,
        scratch_types=[
            pltpu.VMEM((R_MAX, 128), jnp.int32),
            pltpu.VMEM((128,), jnp.float32),
            pltpu.VMEM((RPT,), jnp.float32),
            pltpu.VMEM_SHARED((N_PAD,), jnp.float32),
            pltpu.SemaphoreType.DMA,
        ],
        compiler_params=params,
    )
    scat_call = pl.kernel(
        _scat_body,
        out_type=jax.ShapeDtypeStruct((NC, N_PAD, H), jnp.float32),
        mesh=mesh,
        scratch_types=[
            pltpu.VMEM((2, R_MAX, 128), jnp.int32),
            pltpu.VMEM((2, CHE, H), jnp.float32),
            pltpu.VMEM((RPT, H), jnp.float32),
            pltpu.VMEM_SHARED((N_PAD, H), jnp.float32),
            pltpu.SemaphoreType.DMA,
            pltpu.SemaphoreType.DMA,
        ],
        compiler_params=params,
    )
    return deg_call, scat_call


def _tc1_body(x_ref, w1_ref, degp_ref, hp_ref, dinvb_ref):
    deg = degp_ref[0] + degp_ref[1] + 1.0
    dinv = lax.rsqrt(deg)[:N].reshape(N, 1)
    h = jnp.dot(x_ref[...], w1_ref[...], preferred_element_type=jnp.float32)
    hp_ref[...] = h * dinv
    dinvb_ref[...] = jnp.broadcast_to(dinv, (N, H))


_tc1_call = pl.pallas_call(
    _tc1_body,
    out_shape=[
        jax.ShapeDtypeStruct((N, H), jnp.float32),
        jax.ShapeDtypeStruct((N, H), jnp.float32),
    ],
)


def _tc2_body(acc_ref, hp_ref, dinvp_ref, b1t_ref, w2b_ref, hp2_ref):
    a = acc_ref[...]
    accp = a[:NP8] + a[PADP:PADP + NP8]
    dinvp = dinvp_ref[...]
    x1 = jnp.maximum(dinvp * (accp + hp_ref[...]) + b1t_ref[...], 0.0)
    h2 = jnp.dot(x1, w2b_ref[...], preferred_element_type=jnp.float32)
    hp2_ref[...] = h2 * dinvp


_tc2_call = pl.pallas_call(
    _tc2_body,
    out_shape=jax.ShapeDtypeStruct((NP8, 128), jnp.float32),
)


def _tc3_body(acc_ref, hp2_ref, dinvp_ref, b2t_ref, batch_ref,
              wfc_ref, bfc_ref, wout_ref, bout_ref, out_ref):
    a = acc_ref[...]
    accp = a[:NP8] + a[PADP:PADP + NP8]
    x2p = jnp.maximum(dinvp_ref[...] * (accp + hp2_ref[...]) + b2t_ref[...],
                      0.0)
    bt = batch_ref[...]
    gi = lax.broadcasted_iota(jnp.int32, (NP8, G), 1)
    ones_col = jnp.ones((NP8, 1), jnp.float32)
    dn = (((0,), (0,)), ((), ()))
    seg = jnp.zeros((G, H), jnp.float32)
    cnt = jnp.zeros((G, 1), jnp.float32)
    for j in range(8):
        mj = jnp.where(bt[:, j:j + 1] == gi, 1.0, 0.0)
        xs = x2p[:, H * j:H * (j + 1)]
        seg = seg + lax.dot_general(mj, xs, dn,
                                    preferred_element_type=jnp.float32)
        cnt = cnt + lax.dot_general(mj, ones_col, dn,
                                    preferred_element_type=jnp.float32)
    pooled = seg / jnp.maximum(cnt, 1.0)
    hfc = jnp.maximum(
        jnp.dot(pooled, wfc_ref[...], preferred_element_type=jnp.float32)
        + bfc_ref[...], 0.0)
    out_ref[...] = (jnp.dot(hfc, wout_ref[...],
                            preferred_element_type=jnp.float32) + bout_ref[...])


_tc3_call = pl.pallas_call(
    _tc3_body,
    out_shape=jax.ShapeDtypeStruct((G, 2), jnp.float32),
)


def kernel(features, adj, batch, W1, b1, W2, b2, Wfc, bfc, Wout, bout):
    adjr = adj.reshape(2, ROWS, 128)
    w2big = (jnp.eye(8, dtype=jnp.float32)[:, None, :, None]
             * W2[None, :, None, :]).reshape(8 * H, 8 * H)
    b1t = jnp.tile(b1, 8).reshape(1, 128)
    b2t = jnp.tile(b2, 8).reshape(1, 128)

    deg_call, scat_call = _sc_calls()
    degp = deg_call(adjr)
    hp1, dinvb = _tc1_call(features, W1, degp)
    dinvp = dinvb.reshape(NP8, 128)
    acc1 = scat_call(hp1, adjr)
    hp2p = _tc2_call(acc1.reshape(2 * PADP, 128), hp1.reshape(NP8, 128),
                     dinvp, b1t, w2big)
    acc2 = scat_call(hp2p.reshape(N, H), adjr)
    out = _tc3_call(acc2.reshape(2 * PADP, 128), hp2p, dinvp, b2t,
                    batch.reshape(NP8, 8),
                    Wfc, bfc.reshape(1, 16), Wout, bout.reshape(1, 2))
    return out


# speedup vs baseline: 1.0840x; 1.0414x over previous
import functools

import jax
import jax.numpy as jnp
from jax import lax
from jax.experimental import pallas as pl
from jax.experimental.pallas import tpu as pltpu
from jax.experimental.pallas import tpu_sc as plsc

N = 10000
E = 320000
D = 128
H = 16
G = 128
NC = 2
NS = 16
N_PAD = 10240
ROWS = E // 128
CH = 10
CHUNKS = ROWS // CH
C0_CHUNKS = 125
C1_CHUNKS = CHUNKS - C0_CHUNKS
_Q0, _R0 = divmod(C0_CHUNKS, NS)
_Q1, _R1 = divmod(C1_CHUNKS, NS)
R_MAX = (max(_Q0 + (1 if _R0 else 0), _Q1 + (1 if _R1 else 0))) * CH
CHE = CH * 128
RPT = N_PAD // NS
NP8 = N // 8
PADP = N_PAD * H // 128


def _worker_span(c, s):
    on0 = (c == 0)
    nch = jnp.where(on0, _Q0 + (s < _R0), _Q1 + (s < _R1)).astype(jnp.int32)
    cb = jnp.where(on0, s * _Q0 + jnp.minimum(s, _R0),
                   C0_CHUNKS + s * _Q1 + jnp.minimum(s, _R1)).astype(jnp.int32)
    return cb * CH, nch


def _deg_body(adj_hbm, out_hbm, idx_v, ones_v, zero_v, deg_sh, sem_s):
    c = lax.axis_index("c")
    s = lax.axis_index("s")
    for i in range(8):
        ones_v[pl.ds(i * 16, 16)] = jnp.ones((16,), jnp.float32)

    def zinit(i, carry):
        zero_v[pl.ds(i * 16, 16)] = jnp.zeros((16,), jnp.float32)
        return carry

    lax.fori_loop(0, RPT // 16, zinit, 0)
    pltpu.sync_copy(zero_v, deg_sh.at[pl.ds(s * RPT, RPT)])
    plsc.subcore_barrier()

    base, nch = _worker_span(c, s)
    pstart = jnp.minimum(base, ROWS - R_MAX)
    d = base - pstart
    pltpu.sync_copy(adj_hbm.at[1, pl.ds(pstart, R_MAX)], idx_v)
    drain = pltpu.make_async_copy(
        adj_hbm.at[1, pl.ds(0, CH)], idx_v.at[pl.ds(0, CH)], sem_s)

    def step(j, carry):
        @pl.when(j >= 1)
        def _():
            drain.wait()

        for k in range(CH):
            pltpu.async_copy(ones_v, deg_sh.at[idx_v.at[d + j * CH + k]],
                             sem_s, add=True)
        return carry

    lax.fori_loop(0, nch, step, 0)
    drain.wait()
    plsc.subcore_barrier()
    pltpu.sync_copy(deg_sh.at[pl.ds(s * RPT, RPT)],
                    out_hbm.at[c, pl.ds(s * RPT, RPT)])


def _scat_body(tab_hbm, adj_hbm, out_hbm,
               idx_v, rows_v, zrow_v, acc_sh, sem_g, sem_s):
    c = lax.axis_index("c")
    s = lax.axis_index("s")

    def zinit(i, carry):
        zrow_v[i, :] = jnp.zeros((16,), jnp.float32)
        return carry

    lax.fori_loop(0, RPT, zinit, 0)
    pltpu.sync_copy(zrow_v, acc_sh.at[pl.ds(s * RPT, RPT)])
    plsc.subcore_barrier()

    base, nch = _worker_span(c, s)
    pstart = jnp.minimum(base, ROWS - R_MAX)
    d = base - pstart
    pltpu.sync_copy(adj_hbm.at[0, pl.ds(pstart, R_MAX)], idx_v.at[0])
    pltpu.sync_copy(adj_hbm.at[1, pl.ds(pstart, R_MAX)], idx_v.at[1])

    dummy = tab_hbm.at[pl.ds(0, CHE)]

    def fire_gathers(j, b):
        for k in range(CH):
            pltpu.async_copy(tab_hbm.at[idx_v.at[0, d + j * CH + k]],
                             rows_v.at[b, pl.ds(k * 128, 128)], sem_g)

    def fire_scatters(j, b):
        for k in range(CH):
            pltpu.async_copy(rows_v.at[b, pl.ds(k * 128, 128)],
                             acc_sh.at[idx_v.at[1, d + j * CH + k]],
                             sem_s, add=True)

    fire_gathers(0, 0)

    def step(j, carry):
        b = lax.rem(j, 2)

        @pl.when(j >= 1)
        def _():
            pltpu.make_async_copy(dummy, rows_v.at[1 - b], sem_s).wait()

        @pl.when(j < nch - 1)
        def _():
            fire_gathers(j + 1, 1 - b)

        pltpu.make_async_copy(dummy, rows_v.at[b], sem_g).wait()
        fire_scatters(j, b)
        return carry

    lax.fori_loop(0, nch, step, 0)
    pltpu.make_async_copy(dummy, rows_v.at[lax.rem(nch - 1, 2)], sem_s).wait()
    plsc.subcore_barrier()
    pltpu.sync_copy(acc_sh.at[pl.ds(s * RPT, RPT)],
                    out_hbm.at[c, pl.ds(s * RPT, RPT)])


@functools.cache
def _sc_calls():
    mesh = plsc.VectorSubcoreMesh(core_axis_name="c", subcore_axis_name="s")
    params = pltpu.CompilerParams(use_tc_tiling_on_sc=False)
    deg_call = pl.kernel(
        _deg_body,
        out_type=jax.ShapeDtypeStruct((NC, N_PAD), jnp.float32),
        mesh=mesh,
        scratch_types=[
            pltpu.VMEM((R_MAX, 128), jnp.int32),
            pltpu.VMEM((128,), jnp.float32),
            pltpu.VMEM((RPT,), jnp.float32),
            pltpu.VMEM_SHARED((N_PAD,), jnp.float32),
            pltpu.SemaphoreType.DMA,
        ],
        compiler_params=params,
    )
    scat_call = pl.kernel(
        _scat_body,
        out_type=jax.ShapeDtypeStruct((NC, N_PAD, H), jnp.float32),
        mesh=mesh,
        scratch_types=[
            pltpu.VMEM((2, R_MAX, 128), jnp.int32),
            pltpu.VMEM((2, CHE, H), jnp.float32),
            pltpu.VMEM((RPT, H), jnp.float32),
            pltpu.VMEM_SHARED((N_PAD, H), jnp.float32),
            pltpu.SemaphoreType.DMA,
            pltpu.SemaphoreType.DMA,
        ],
        compiler_params=params,
    )
    return deg_call, scat_call


def _tc1_body(x_ref, w1_ref, degp_ref, hp_ref, dinvb_ref):
    deg = degp_ref[0] + degp_ref[1] + 1.0
    dinv = lax.rsqrt(deg)[:N].reshape(N, 1)
    h = jnp.dot(x_ref[...], w1_ref[...], preferred_element_type=jnp.float32)
    hp_ref[...] = h * dinv
    dinvb_ref[...] = jnp.broadcast_to(dinv, (N, H))


_tc1_call = pl.pallas_call(
    _tc1_body,
    out_shape=[
        jax.ShapeDtypeStruct((N, H), jnp.float32),
        jax.ShapeDtypeStruct((N, H), jnp.float32),
    ],
)


def _tc2_body(acc_ref, hp_ref, dinvp_ref, b1t_ref, w2b_ref, hp2_ref):
    a = acc_ref[...]
    accp = a[:NP8] + a[PADP:PADP + NP8]
    dinvp = dinvp_ref[...]
    x1 = jnp.maximum(dinvp * (accp + hp_ref[...]) + b1t_ref[...], 0.0)
    h2 = jnp.dot(x1, w2b_ref[...], preferred_element_type=jnp.float32)
    hp2_ref[...] = h2 * dinvp


_tc2_call = pl.pallas_call(
    _tc2_body,
    out_shape=jax.ShapeDtypeStruct((NP8, 128), jnp.float32),
)


def _tc3_body(acc_ref, hp2_ref, dinvp_ref, b2t_ref, batch_ref,
              wfc_ref, bfc_ref, wout_ref, bout_ref, out_ref):
    a = acc_ref[...]
    accp = a[:NP8] + a[PADP:PADP + NP8]
    x2p = jnp.maximum(dinvp_ref[...] * (accp + hp2_ref[...]) + b2t_ref[...],
                      0.0)
    bt = batch_ref[...]
    gi = lax.broadcasted_iota(jnp.int32, (NP8, G), 1)
    ones_col = jnp.ones((NP8, 1), jnp.float32)
    dn = (((0,), (0,)), ((), ()))
    seg = jnp.zeros((G, H), jnp.float32)
    cnt = jnp.zeros((G, 1), jnp.float32)
    for j in range(8):
        mj = jnp.where(bt[:, j:j + 1] == gi, 1.0, 0.0)
        xs = x2p[:, H * j:H * (j + 1)]
        seg = seg + lax.dot_general(mj, xs, dn,
                                    preferred_element_type=jnp.float32)
        cnt = cnt + lax.dot_general(mj, ones_col, dn,
                                    preferred_element_type=jnp.float32)
    pooled = seg / jnp.maximum(cnt, 1.0)
    hfc = jnp.maximum(
        jnp.dot(pooled, wfc_ref[...], preferred_element_type=jnp.float32)
        + bfc_ref[...], 0.0)
    out_ref[...] = (jnp.dot(hfc, wout_ref[...],
                            preferred_element_type=jnp.float32) + bout_ref[...])


_tc3_call = pl.pallas_call(
    _tc3_body,
    out_shape=jax.ShapeDtypeStruct((G, 2), jnp.float32),
)


def kernel(features, adj, batch, W1, b1, W2, b2, Wfc, bfc, Wout, bout):
    adjr = adj.reshape(2, ROWS, 128)
    w2big = (jnp.eye(8, dtype=jnp.float32)[:, None, :, None]
             * W2[None, :, None, :]).reshape(8 * H, 8 * H)
    b1t = jnp.tile(b1, 8).reshape(1, 128)
    b2t = jnp.tile(b2, 8).reshape(1, 128)

    deg_call, scat_call = _sc_calls()
    degp = deg_call(adjr)
    hp1, dinvb = _tc1_call(features, W1, degp)
    dinvp = dinvb.reshape(NP8, 128)
    acc1 = scat_call(hp1, adjr)
    hp2p = _tc2_call(acc1.reshape(2 * PADP, 128), hp1.reshape(NP8, 128),
                     dinvp, b1t, w2big)
    acc2 = scat_call(hp2p.reshape(N, H), adjr)
    out = _tc3_call(acc2.reshape(2 * PADP, 128), hp2p, dinvp, b2t,
                    batch.reshape(NP8, 8),
                    Wfc, bfc.reshape(1, 16), Wout, bout.reshape(1, 2))
    return out


# speedup vs baseline: 1.0895x; 1.0050x over previous
import functools

import jax
import jax.numpy as jnp
from jax import lax
from jax.experimental import pallas as pl
from jax.experimental.pallas import tpu as pltpu
from jax.experimental.pallas import tpu_sc as plsc

N = 10000
E = 320000
D = 128
H = 16
G = 128
NC = 2
NS = 16
N_PAD = 10240
IW = 256
ROWS = E // IW
CH = 5
CHUNKS = ROWS // CH
C0_CHUNKS = 125
C1_CHUNKS = CHUNKS - C0_CHUNKS
_Q0, _R0 = divmod(C0_CHUNKS, NS)
_Q1, _R1 = divmod(C1_CHUNKS, NS)
R_MAX = (max(_Q0 + (1 if _R0 else 0), _Q1 + (1 if _R1 else 0))) * CH
CHE = CH * IW
RPT = N_PAD // NS
NP8 = N // 8
PADP = N_PAD * H // 128


def _worker_span(c, s):
    on0 = (c == 0)
    nch = jnp.where(on0, _Q0 + (s < _R0), _Q1 + (s < _R1)).astype(jnp.int32)
    cb = jnp.where(on0, s * _Q0 + jnp.minimum(s, _R0),
                   C0_CHUNKS + s * _Q1 + jnp.minimum(s, _R1)).astype(jnp.int32)
    return cb * CH, nch


def _deg_body(adj_hbm, out_hbm, idx_v, ones_v, zero_v, deg_sh, sem_s):
    c = lax.axis_index("c")
    s = lax.axis_index("s")
    for i in range(IW // 16):
        ones_v[pl.ds(i * 16, 16)] = jnp.ones((16,), jnp.float32)

    def zinit(i, carry):
        zero_v[pl.ds(i * 16, 16)] = jnp.zeros((16,), jnp.float32)
        return carry

    lax.fori_loop(0, RPT // 16, zinit, 0)
    pltpu.sync_copy(zero_v, deg_sh.at[pl.ds(s * RPT, RPT)])
    plsc.subcore_barrier()

    base, nch = _worker_span(c, s)
    pstart = jnp.minimum(base, ROWS - R_MAX)
    d = base - pstart
    pltpu.sync_copy(adj_hbm.at[1, pl.ds(pstart, R_MAX)], idx_v)
    drain = pltpu.make_async_copy(
        adj_hbm.at[1, pl.ds(0, CH)], idx_v.at[pl.ds(0, CH)], sem_s)

    def step(j, carry):
        @pl.when(j >= 1)
        def _():
            drain.wait()

        for k in range(CH):
            pltpu.async_copy(ones_v, deg_sh.at[idx_v.at[d + j * CH + k]],
                             sem_s, add=True)
        return carry

    lax.fori_loop(0, nch, step, 0)
    drain.wait()
    plsc.subcore_barrier()
    pltpu.sync_copy(deg_sh.at[pl.ds(s * RPT, RPT)],
                    out_hbm.at[c, pl.ds(s * RPT, RPT)])


def _scat_body(tab_hbm, adj_hbm, out_hbm,
               idx_v, rows_v, zrow_v, acc_sh, sem_g, sem_s):
    c = lax.axis_index("c")
    s = lax.axis_index("s")

    def zinit(i, carry):
        zrow_v[i, :] = jnp.zeros((16,), jnp.float32)
        return carry

    lax.fori_loop(0, RPT, zinit, 0)
    pltpu.sync_copy(zrow_v, acc_sh.at[pl.ds(s * RPT, RPT)])
    plsc.subcore_barrier()

    base, nch = _worker_span(c, s)
    pstart = jnp.minimum(base, ROWS - R_MAX)
    d = base - pstart
    pltpu.sync_copy(adj_hbm.at[0, pl.ds(pstart, R_MAX)], idx_v.at[0])
    pltpu.sync_copy(adj_hbm.at[1, pl.ds(pstart, R_MAX)], idx_v.at[1])

    dummy = tab_hbm.at[pl.ds(0, CHE)]

    def fire_gathers(j, b):
        for k in range(CH):
            pltpu.async_copy(tab_hbm.at[idx_v.at[0, d + j * CH + k]],
                             rows_v.at[b, pl.ds(k * IW, IW)], sem_g)

    def fire_scatters(j, b):
        for k in range(CH):
            pltpu.async_copy(rows_v.at[b, pl.ds(k * IW, IW)],
                             acc_sh.at[idx_v.at[1, d + j * CH + k]],
                             sem_s, add=True)

    fire_gathers(0, 0)

    def step(j, carry):
        b = lax.rem(j, 2)

        @pl.when(j >= 1)
        def _():
            pltpu.make_async_copy(dummy, rows_v.at[1 - b], sem_s).wait()

        @pl.when(j < nch - 1)
        def _():
            fire_gathers(j + 1, 1 - b)

        pltpu.make_async_copy(dummy, rows_v.at[b], sem_g).wait()
        fire_scatters(j, b)
        return carry

    lax.fori_loop(0, nch, step, 0)
    pltpu.make_async_copy(dummy, rows_v.at[lax.rem(nch - 1, 2)], sem_s).wait()
    plsc.subcore_barrier()
    pltpu.sync_copy(acc_sh.at[pl.ds(s * RPT, RPT)],
                    out_hbm.at[c, pl.ds(s * RPT, RPT)])


@functools.cache
def _sc_calls():
    mesh = plsc.VectorSubcoreMesh(core_axis_name="c", subcore_axis_name="s")
    params = pltpu.CompilerParams(use_tc_tiling_on_sc=False)
    deg_call = pl.kernel(
        _deg_body,
        out_type=jax.ShapeDtypeStruct((NC, N_PAD), jnp.float32),
        mesh=mesh,
        scratch_types=[
            pltpu.VMEM((R_MAX, IW), jnp.int32),
            pltpu.VMEM((IW,), jnp.float32),
            pltpu.VMEM((RPT,), jnp.float32),
            pltpu.VMEM_SHARED((N_PAD,), jnp.float32),
            pltpu.SemaphoreType.DMA,
        ],
        compiler_params=params,
    )
    scat_call = pl.kernel(
        _scat_body,
        out_type=jax.ShapeDtypeStruct((NC, N_PAD, H), jnp.float32),
        mesh=mesh,
        scratch_types=[
            pltpu.VMEM((2, R_MAX, IW), jnp.int32),
            pltpu.VMEM((2, CHE, H), jnp.float32),
            pltpu.VMEM((RPT, H), jnp.float32),
            pltpu.VMEM_SHARED((N_PAD, H), jnp.float32),
            pltpu.SemaphoreType.DMA,
            pltpu.SemaphoreType.DMA,
        ],
        compiler_params=params,
    )
    return deg_call, scat_call


def _tc1_body(x_ref, w1_ref, degp_ref, hp_ref, dinvb_ref):
    deg = degp_ref[0] + degp_ref[1] + 1.0
    dinv = lax.rsqrt(deg)[:N].reshape(N, 1)
    h = jnp.dot(x_ref[...], w1_ref[...], preferred_element_type=jnp.float32)
    hp_ref[...] = h * dinv
    dinvb_ref[...] = jnp.broadcast_to(dinv, (N, H))


_tc1_call = pl.pallas_call(
    _tc1_body,
    out_shape=[
        jax.ShapeDtypeStruct((N, H), jnp.float32),
        jax.ShapeDtypeStruct((N, H), jnp.float32),
    ],
)


def _tc2_body(acc_ref, hp_ref, dinvp_ref, b1t_ref, w2b_ref, hp2_ref):
    a = acc_ref[...]
    accp = a[:NP8] + a[PADP:PADP + NP8]
    dinvp = dinvp_ref[...]
    x1 = jnp.maximum(dinvp * (accp + hp_ref[...]) + b1t_ref[...], 0.0)
    h2 = jnp.dot(x1, w2b_ref[...], preferred_element_type=jnp.float32)
    hp2_ref[...] = h2 * dinvp


_tc2_call = pl.pallas_call(
    _tc2_body,
    out_shape=jax.ShapeDtypeStruct((NP8, 128), jnp.float32),
)


def _tc3_body(acc_ref, hp2_ref, dinvp_ref, b2t_ref, batch_ref,
              wfc_ref, bfc_ref, wout_ref, bout_ref, out_ref):
    a = acc_ref[...]
    accp = a[:NP8] + a[PADP:PADP + NP8]
    x2p = jnp.maximum(dinvp_ref[...] * (accp + hp2_ref[...]) + b2t_ref[...],
                      0.0)
    bt = batch_ref[...]
    gi = lax.broadcasted_iota(jnp.int32, (NP8, G), 1)
    ones_col = jnp.ones((NP8, 1), jnp.float32)
    dn = (((0,), (0,)), ((), ()))
    seg = jnp.zeros((G, H), jnp.float32)
    cnt = jnp.zeros((G, 1), jnp.float32)
    for j in range(8):
        mj = jnp.where(bt[:, j:j + 1] == gi, 1.0, 0.0)
        xs = x2p[:, H * j:H * (j + 1)]
        seg = seg + lax.dot_general(mj, xs, dn,
                                    preferred_element_type=jnp.float32)
        cnt = cnt + lax.dot_general(mj, ones_col, dn,
                                    preferred_element_type=jnp.float32)
    pooled = seg / jnp.maximum(cnt, 1.0)
    hfc = jnp.maximum(
        jnp.dot(pooled, wfc_ref[...], preferred_element_type=jnp.float32)
        + bfc_ref[...], 0.0)
    out_ref[...] = (jnp.dot(hfc, wout_ref[...],
                            preferred_element_type=jnp.float32) + bout_ref[...])


_tc3_call = pl.pallas_call(
    _tc3_body,
    out_shape=jax.ShapeDtypeStruct((G, 2), jnp.float32),
)


def kernel(features, adj, batch, W1, b1, W2, b2, Wfc, bfc, Wout, bout):
    adjr = adj.reshape(2, ROWS, IW)
    w2big = (jnp.eye(8, dtype=jnp.float32)[:, None, :, None]
             * W2[None, :, None, :]).reshape(8 * H, 8 * H)
    b1t = jnp.tile(b1, 8).reshape(1, 128)
    b2t = jnp.tile(b2, 8).reshape(1, 128)

    deg_call, scat_call = _sc_calls()
    degp = deg_call(adjr)
    hp1, dinvb = _tc1_call(features, W1, degp)
    dinvp = dinvb.reshape(NP8, 128)
    acc1 = scat_call(hp1, adjr)
    hp2p = _tc2_call(acc1.reshape(2 * PADP, 128), hp1.reshape(NP8, 128),
                     dinvp, b1t, w2big)
    acc2 = scat_call(hp2p.reshape(N, H), adjr)
    out = _tc3_call(acc2.reshape(2 * PADP, 128), hp2p, dinvp, b2t,
                    batch.reshape(NP8, 8),
                    Wfc, bfc.reshape(1, 16), Wout, bout.reshape(1, 2))
    return out
